# baseline (speedup 1.0000x reference)
import functools

import jax
import jax.numpy as jnp
from jax import lax
from jax.experimental import pallas as pl
from jax.experimental.pallas import tpu as pltpu
from jax.experimental.pallas import tpu_sc as plsc

_ATOM_DIMS = (119, 4, 12, 12, 10, 6, 6, 2, 2)
_BOND_DIMS = (5, 6, 2)
_ATOM_PAD = 176
_BOND_PAD = 16
_DIM_H = 112
_DIM_EMB = 128

_NB = 1000
_EB = 3200
_C_ROWS = 64

_CH = 400
_NW = 32


def _prep_body(ea_ref, tbl_ref, codes_ref, c_ref):
    ea = ea_ref[...]
    code_col = ea[:, 0:1] * 12 + ea[:, 1:2] * 2 + ea[:, 2:3]
    codes_ref[pl.ds(pl.program_id(0) * _EB, _EB)] = jnp.reshape(code_col, (_EB,))

    @pl.when(pl.program_id(0) == 0)
    def _():
        k = jax.lax.broadcasted_iota(jnp.int32, (_C_ROWS, 1), 0)
        iota = jax.lax.broadcasted_iota(jnp.int32, (_C_ROWS, _BOND_PAD), 1)
        mh = ((iota == k // 12).astype(jnp.float32)
              + (iota == 5 + (k % 12) // 2).astype(jnp.float32)
              + (iota == 11 + k % 2).astype(jnp.float32))
        c_ref[...] = jnp.dot(mh, tbl_ref[...], preferred_element_type=jnp.float32)


def _node_body(x_ref, pe_ref, tbl_ref, w1_ref, b1_ref, w2_ref, b2_ref, out_ref):
    xb = x_ref[...]
    iota = jax.lax.broadcasted_iota(jnp.int32, (_NB, _ATOM_PAD), 1)
    mh = jnp.zeros((_NB, _ATOM_PAD), jnp.float32)
    off = 0
    for c, d in enumerate(_ATOM_DIMS):
        mh = mh + (iota == xb[:, c:c + 1] + off).astype(jnp.float32)
        off += d
    h = jnp.dot(mh, tbl_ref[...], preferred_element_type=jnp.float32)
    p = jnp.maximum(jnp.dot(pe_ref[...], w1_ref[...],
                            preferred_element_type=jnp.float32) + b1_ref[...], 0.0)
    p = jnp.maximum(jnp.dot(p, w2_ref[...],
                            preferred_element_type=jnp.float32) + b2_ref[...], 0.0)
    out_ref[...] = jnp.concatenate([h, p], axis=1)


def _sc_edge_body(codes_hbm, c_hbm, out_hbm, codes_v, rows_v, c_sh):
    E = codes_hbm.shape[0]
    bpw = E // _NW
    wid = lax.axis_index("s") * 2 + lax.axis_index("c")
    base = wid * bpw

    @pl.when(lax.axis_index("s") == 0)
    def _():
        pltpu.sync_copy(c_hbm, c_sh)

    plsc.subcore_barrier()

    def chunk_body(k, _):
        row0 = base + k * _CH
        pltpu.sync_copy(codes_hbm.at[pl.ds(row0, _CH)], codes_v)
        pltpu.sync_copy(c_sh.at[codes_v], rows_v)
        pltpu.sync_copy(rows_v, out_hbm.at[pl.ds(row0, _CH)])
        return 0

    lax.fori_loop(0, bpw // _CH, chunk_body, 0)


def kernel(x, edge_attr, pestat_RWSE, atom_tables, bond_tables, W1, b1, W2, b2):
    N = x.shape[0]
    E = edge_attr.shape[0]

    atbl = jnp.concatenate(list(atom_tables), axis=0)
    atbl = jnp.pad(atbl, ((0, _ATOM_PAD - atbl.shape[0]), (0, 0)))
    btbl = jnp.concatenate(list(bond_tables), axis=0)
    btbl = jnp.pad(btbl, ((0, _BOND_PAD - btbl.shape[0]), (0, 0)))

    codes, ctbl = pl.pallas_call(
        _prep_body,
        grid=(E // _EB,),
        in_specs=[
            pl.BlockSpec((_EB, 3), lambda i: (i, 0)),
            pl.BlockSpec((_BOND_PAD, _DIM_EMB), lambda i: (0, 0)),
        ],
        out_specs=[
            pl.BlockSpec((320000,), lambda i: (0,)),
            pl.BlockSpec((_C_ROWS, _DIM_EMB), lambda i: (0, 0)),
        ],
        out_shape=[
            jax.ShapeDtypeStruct((E,), jnp.int32),
            jax.ShapeDtypeStruct((_C_ROWS, _DIM_EMB), jnp.float32),
        ],
    )(edge_attr, btbl)

    sc_edge = functools.partial(
        pl.kernel,
        mesh=plsc.VectorSubcoreMesh(core_axis_name="c", subcore_axis_name="s"),
        out_type=jax.ShapeDtypeStruct((E, _DIM_EMB), jnp.float32),
        scratch_types=[
            pltpu.VMEM((_CH,), jnp.int32),
            pltpu.VMEM((_CH, _DIM_EMB), jnp.float32),
            pltpu.VMEM_SHARED((_C_ROWS, _DIM_EMB), jnp.float32),
        ],
    )(_sc_edge_body)
    e = sc_edge(codes, ctbl)

    new_x = pl.pallas_call(
        _node_body,
        grid=(N // _NB,),
        in_specs=[
            pl.BlockSpec((_NB, 9), lambda i: (i, 0)),
            pl.BlockSpec((_NB, 20), lambda i: (i, 0)),
            pl.BlockSpec((_ATOM_PAD, _DIM_H), lambda i: (0, 0)),
            pl.BlockSpec((20, 32), lambda i: (0, 0)),
            pl.BlockSpec((1, 32), lambda i: (0, 0)),
            pl.BlockSpec((32, 16), lambda i: (0, 0)),
            pl.BlockSpec((1, 16), lambda i: (0, 0)),
        ],
        out_specs=pl.BlockSpec((_NB, _DIM_EMB), lambda i: (i, 0)),
        out_shape=jax.ShapeDtypeStruct((N, _DIM_EMB), jnp.float32),
    )(x, pestat_RWSE, atbl, W1, b1.reshape(1, 32), W2, b2.reshape(1, 16))

    return new_x, e


# speedup vs baseline: 1.6196x; 1.6196x over previous
import jax
import jax.numpy as jnp
from jax.experimental import pallas as pl

_ATOM_DIMS = (119, 4, 12, 12, 10, 6, 6, 2, 2)
_BOND_DIMS = (5, 6, 2)
_ATOM_PAD = 176
_BOND_PAD = 16
_DIM_H = 112
_DIM_EMB = 128

_EB = 6400
_NB = 1000


def _edge_body_probe(tbl_ref, out_ref):
    out_ref[...] = jnp.broadcast_to(tbl_ref[0:1, :], (_EB, 128))


def _edge_body(ea_ref, tbl_ref, out_ref):
    ea = ea_ref[...]
    iota = jax.lax.broadcasted_iota(jnp.int32, (_EB, _BOND_PAD), 1)
    mh = jnp.zeros((_EB, _BOND_PAD), jnp.float32)
    off = 0
    for c, d in enumerate(_BOND_DIMS):
        mh = mh + (iota == ea[:, c:c + 1] + off).astype(jnp.float32)
        off += d
    out_ref[...] = jnp.dot(mh, tbl_ref[...], preferred_element_type=jnp.float32)


def _node_body(x_ref, pe_ref, tbl_ref, w1_ref, b1_ref, w2_ref, b2_ref, out_ref):
    xb = x_ref[...]
    iota = jax.lax.broadcasted_iota(jnp.int32, (_NB, _ATOM_PAD), 1)
    mh = jnp.zeros((_NB, _ATOM_PAD), jnp.float32)
    off = 0
    for c, d in enumerate(_ATOM_DIMS):
        mh = mh + (iota == xb[:, c:c + 1] + off).astype(jnp.float32)
        off += d
    h = jnp.dot(mh, tbl_ref[...], preferred_element_type=jnp.float32)
    p = jnp.maximum(jnp.dot(pe_ref[...], w1_ref[...],
                            preferred_element_type=jnp.float32) + b1_ref[...], 0.0)
    p = jnp.maximum(jnp.dot(p, w2_ref[...],
                            preferred_element_type=jnp.float32) + b2_ref[...], 0.0)
    out_ref[...] = jnp.concatenate([h, p], axis=1)


def kernel(x, edge_attr, pestat_RWSE, atom_tables, bond_tables, W1, b1, W2, b2):
    N = x.shape[0]
    E = edge_attr.shape[0]

    atbl = jnp.concatenate(list(atom_tables), axis=0)
    atbl = jnp.pad(atbl, ((0, _ATOM_PAD - atbl.shape[0]), (0, 0)))
    btbl = jnp.concatenate(list(bond_tables), axis=0)
    btbl = jnp.pad(btbl, ((0, _BOND_PAD - btbl.shape[0]), (0, 0)))

    e = pl.pallas_call(
        _edge_body,
        grid=(E // _EB,),
        in_specs=[
            pl.BlockSpec((_EB, 3), lambda i: (i, 0)),
            pl.BlockSpec((_BOND_PAD, _DIM_EMB), lambda i: (0, 0)),
        ],
        out_specs=pl.BlockSpec((_EB, _DIM_EMB), lambda i: (i, 0)),
        out_shape=jax.ShapeDtypeStruct((E, _DIM_EMB), jnp.float32),
    )(edge_attr, btbl)

    new_x = pl.pallas_call(
        _node_body,
        grid=(N // _NB,),
        in_specs=[
            pl.BlockSpec((_NB, 9), lambda i: (i, 0)),
            pl.BlockSpec((_NB, 20), lambda i: (i, 0)),
            pl.BlockSpec((_ATOM_PAD, _DIM_H), lambda i: (0, 0)),
            pl.BlockSpec((20, 32), lambda i: (0, 0)),
            pl.BlockSpec((1, 32), lambda i: (0, 0)),
            pl.BlockSpec((32, 16), lambda i: (0, 0)),
            pl.BlockSpec((1, 16), lambda i: (0, 0)),
        ],
        out_specs=pl.BlockSpec((_NB, _DIM_EMB), lambda i: (i, 0)),
        out_shape=jax.ShapeDtypeStruct((N, _DIM_EMB), jnp.float32),
    )(x, pestat_RWSE, atbl, W1, b1.reshape(1, 32), W2, b2.reshape(1, 16))

    return new_x, e


# speedup vs baseline: 1.7072x; 1.0540x over previous
import jax
import jax.numpy as jnp
from jax.experimental import pallas as pl

_ATOM_DIMS = (119, 4, 12, 12, 10, 6, 6, 2, 2)
_BOND_DIMS = (5, 6, 2)
_ATOM_PAD = 176
_BOND_PAD = 16
_DIM_H = 112
_DIM_EMB = 128

_EB = 12800
_NB = 1000


def _edge_body_probe(tbl_ref, out_ref):
    out_ref[...] = jnp.broadcast_to(tbl_ref[0:1, :], (_EB, 128))


def _edge_body(ea_ref, tbl_ref, out_ref):
    ea = ea_ref[...]
    iota = jax.lax.broadcasted_iota(jnp.int32, (_EB, _BOND_PAD), 1)
    mh = jnp.zeros((_EB, _BOND_PAD), jnp.float32)
    off = 0
    for c, d in enumerate(_BOND_DIMS):
        mh = mh + (iota == ea[:, c:c + 1] + off).astype(jnp.float32)
        off += d
    out_ref[...] = jnp.dot(mh, tbl_ref[...], preferred_element_type=jnp.float32)


def _node_body(x_ref, pe_ref, tbl_ref, w1_ref, b1_ref, w2_ref, b2_ref, out_ref):
    xb = x_ref[...]
    iota = jax.lax.broadcasted_iota(jnp.int32, (_NB, _ATOM_PAD), 1)
    mh = jnp.zeros((_NB, _ATOM_PAD), jnp.float32)
    off = 0
    for c, d in enumerate(_ATOM_DIMS):
        mh = mh + (iota == xb[:, c:c + 1] + off).astype(jnp.float32)
        off += d
    h = jnp.dot(mh, tbl_ref[...], preferred_element_type=jnp.float32)
    p = jnp.maximum(jnp.dot(pe_ref[...], w1_ref[...],
                            preferred_element_type=jnp.float32) + b1_ref[...], 0.0)
    p = jnp.maximum(jnp.dot(p, w2_ref[...],
                            preferred_element_type=jnp.float32) + b2_ref[...], 0.0)
    out_ref[...] = jnp.concatenate([h, p], axis=1)


def kernel(x, edge_attr, pestat_RWSE, atom_tables, bond_tables, W1, b1, W2, b2):
    N = x.shape[0]
    E = edge_attr.shape[0]

    atbl = jnp.concatenate(list(atom_tables), axis=0)
    atbl = jnp.pad(atbl, ((0, _ATOM_PAD - atbl.shape[0]), (0, 0)))
    btbl = jnp.concatenate(list(bond_tables), axis=0)
    btbl = jnp.pad(btbl, ((0, _BOND_PAD - btbl.shape[0]), (0, 0)))

    e = pl.pallas_call(
        _edge_body,
        grid=(E // _EB,),
        in_specs=[
            pl.BlockSpec((_EB, 3), lambda i: (i, 0)),
            pl.BlockSpec((_BOND_PAD, _DIM_EMB), lambda i: (0, 0)),
        ],
        out_specs=pl.BlockSpec((_EB, _DIM_EMB), lambda i: (i, 0)),
        out_shape=jax.ShapeDtypeStruct((E, _DIM_EMB), jnp.float32),
    )(edge_attr, btbl)

    new_x = pl.pallas_call(
        _node_body,
        grid=(N // _NB,),
        in_specs=[
            pl.BlockSpec((_NB, 9), lambda i: (i, 0)),
            pl.BlockSpec((_NB, 20), lambda i: (i, 0)),
            pl.BlockSpec((_ATOM_PAD, _DIM_H), lambda i: (0, 0)),
            pl.BlockSpec((20, 32), lambda i: (0, 0)),
            pl.BlockSpec((1, 32), lambda i: (0, 0)),
            pl.BlockSpec((32, 16), lambda i: (0, 0)),
            pl.BlockSpec((1, 16), lambda i: (0, 0)),
        ],
        out_specs=pl.BlockSpec((_NB, _DIM_EMB), lambda i: (i, 0)),
        out_shape=jax.ShapeDtypeStruct((N, _DIM_EMB), jnp.float32),
    )(x, pestat_RWSE, atbl, W1, b1.reshape(1, 32), W2, b2.reshape(1, 16))

    return new_x, e
